# Initial kernel scaffold; baseline (speedup 1.0000x reference)
import jax
import jax.numpy as jnp
from jax.experimental import pallas as pl


def kernel(src_V, src_E, dist_grid, ref_edge_vec):
    raise NotImplementedError("write your pallas kernel here")


# speedup vs baseline: 1.4439x; 1.4439x over previous
import functools

import jax
import jax.numpy as jnp
from jax import lax
from jax.experimental import pallas as pl
from jax.experimental.pallas import tpu as pltpu
from jax.experimental.pallas import tpu_sc as plsc

NC = 2
NS = 16
NW = NC * NS
LANES = 16
GRID_R = 64


@functools.partial(jax.jit, static_argnums=(8, 9, 10))
def _sc_loss(vrows, e0, e1, reff, gridf, vxh, vyh, vzh, N, C, CV):
    E = e0.shape[0]
    EPT = E // NW
    NCH = EPT // C

    mesh = plsc.VectorSubcoreMesh(
        core_axis_name="c", subcore_axis_name="s",
        num_cores=NC, num_subcores=NS)

    scratch = [
        pltpu.VMEM((C,), jnp.int32),
        pltpu.VMEM((C,), jnp.int32),
        pltpu.VMEM((C, 3), jnp.float32),
        pltpu.VMEM((C, 3), jnp.float32),
        pltpu.VMEM((3 * C,), jnp.float32),
        pltpu.VMEM((CV,), jnp.float32),
        pltpu.VMEM((CV,), jnp.float32),
        pltpu.VMEM((CV,), jnp.float32),
        pltpu.VMEM((CV,), jnp.float32),
        pltpu.VMEM((CV,), jnp.float32),
        pltpu.VMEM((CV,), jnp.float32),
    ]
    scratch += [pltpu.VMEM((CV,), jnp.int32) for _ in range(8)]
    scratch += [pltpu.VMEM((CV,), jnp.float32) for _ in range(8)]
    scratch += [
        pltpu.VMEM((LANES,), jnp.float32),
        pltpu.SemaphoreType.DMA,
        pltpu.SemaphoreType.DMA,
        pltpu.SemaphoreType.DMA,
    ]

    @functools.partial(
        pl.kernel,
        out_type=jax.ShapeDtypeStruct((NW, LANES), jnp.float32),
        mesh=mesh,
        scratch_types=scratch,
        compiler_params=pltpu.CompilerParams(
            needs_layout_passes=False, use_tc_tiling_on_sc=False),
    )
    def k(vrows_h, e0_h, e1_h, reff_h, gridf_h, vx_h, vy_h, vz_h, out_h, *scr):
        (idx0_v, idx1_v, rows0_v, rows1_v, ref_v,
         vx_v, vy_v, vz_v, fx_v, fy_v, fz_v) = scr[:11]
        cidx = scr[11:19]
        cval = scr[19:27]
        acc_v = scr[27]
        sem0, sem1, sem2 = scr[28:31]

        wid = lax.axis_index("s") * NC + lax.axis_index("c")
        iot = lax.iota(jnp.int32, LANES)

        eb = wid * EPT

        def echunk(i, acc):
            base = eb + i * C
            pltpu.sync_copy(e0_h.at[pl.ds(base, C)], idx0_v)
            pltpu.sync_copy(e1_h.at[pl.ds(base, C)], idx1_v)
            cp0 = pltpu.async_copy(vrows_h.at[idx0_v], rows0_v, sem0)
            cp1 = pltpu.async_copy(vrows_h.at[idx1_v], rows1_v, sem1)
            cp2 = pltpu.async_copy(reff_h.at[pl.ds(3 * base, 3 * C)], ref_v, sem2)
            cp0.wait()
            cp1.wait()
            cp2.wait()

            def grp(g, a):
                r = g * LANES + iot
                row = (r * 43691) >> 17
                col = r - row * 3
                av = plsc.load_gather(rows0_v, [row, col])
                bv = plsc.load_gather(rows1_v, [row, col])
                rv = ref_v[pl.ds(g * LANES, LANES)]
                d = av - bv - rv
                return a + d * d

            return lax.fori_loop(0, (3 * C) // LANES, grp, acc)

        acc = lax.fori_loop(0, NCH, echunk, jnp.zeros((LANES,), jnp.float32))

        vb = wid * CV
        pltpu.sync_copy(vx_h.at[pl.ds(vb, CV)], vx_v)
        pltpu.sync_copy(vy_h.at[pl.ds(vb, CV)], vy_v)
        pltpu.sync_copy(vz_h.at[pl.ds(vb, CV)], vz_v)

        def vprep(g, _):
            sl = pl.ds(g * LANES, LANES)

            def prep(p):
                u = jnp.minimum(
                    jnp.maximum((p + 1.0) * 0.5 * float(GRID_R - 1), 0.0),
                    float(GRID_R - 1) - 1e-4)
                i0 = u.astype(jnp.int32)
                return i0, u - i0.astype(jnp.float32)

            x0, fx = prep(vx_v[sl])
            y0, fy = prep(vy_v[sl])
            z0, fz = prep(vz_v[sl])
            b = x0 * (GRID_R * GRID_R) + y0 * GRID_R + z0
            cidx[0][sl] = b
            cidx[1][sl] = b + 1
            cidx[2][sl] = b + GRID_R
            cidx[3][sl] = b + GRID_R + 1
            cidx[4][sl] = b + GRID_R * GRID_R
            cidx[5][sl] = b + GRID_R * GRID_R + 1
            cidx[6][sl] = b + GRID_R * GRID_R + GRID_R
            cidx[7][sl] = b + GRID_R * GRID_R + GRID_R + 1
            fx_v[sl] = fx
            fy_v[sl] = fy
            fz_v[sl] = fz
            return _

        lax.fori_loop(0, CV // LANES, vprep, 0)

        cps = [pltpu.async_copy(gridf_h.at[cidx[j]], cval[j], sem0)
               for j in range(8)]
        for cp in cps:
            cp.wait()

        def vgrp(g, a):
            sl = pl.ds(g * LANES, LANES)
            fx = fx_v[sl]
            fy = fy_v[sl]
            fz = fz_v[sl]
            c00 = cval[0][sl] * (1 - fx) + cval[4][sl] * fx
            c10 = cval[2][sl] * (1 - fx) + cval[6][sl] * fx
            c01 = cval[1][sl] * (1 - fx) + cval[5][sl] * fx
            c11 = cval[3][sl] * (1 - fx) + cval[7][sl] * fx
            c0 = c00 * (1 - fy) + c10 * fy
            c1 = c01 * (1 - fy) + c11 * fy
            d = c0 * (1 - fz) + c1 * fz
            vid = vb + g * LANES + iot
            dm = jnp.where(vid < N, d, 0.0)
            return a + dm * dm

        acc = lax.fori_loop(0, CV // LANES, vgrp, acc)

        acc_v[...] = acc
        pltpu.sync_copy(acc_v, out_h.at[wid])

    return k(vrows, e0, e1, reff, gridf, vxh, vyh, vzh)


def kernel(src_V, src_E, dist_grid, ref_edge_vec):
    N = src_V.shape[0]
    E = src_E.shape[0]
    assert E % NW == 0
    ept = E // NW
    C = 16
    for cand in range(16, 4097, 16):
        if ept % cand == 0:
            C = cand
    align = LANES * NW
    NP = ((N + align - 1) // align) * align
    CV = NP // NW

    e0 = src_E[:, 0]
    e1 = src_E[:, 1]
    reff = ref_edge_vec.reshape(-1)
    gridf = dist_grid.reshape(-1)
    pad = NP - N
    vx = jnp.pad(src_V[:, 0], (0, pad))
    vy = jnp.pad(src_V[:, 1], (0, pad))
    vz = jnp.pad(src_V[:, 2], (0, pad))

    out = _sc_loss(src_V, e0, e1, reff, gridf, vx, vy, vz, N, C, CV)
    return 0.5 * jnp.sum(out)


# speedup vs baseline: 11.1508x; 7.7227x over previous
import functools

import jax
import jax.numpy as jnp
from jax import lax
from jax.experimental import pallas as pl
from jax.experimental.pallas import tpu as pltpu
from jax.experimental.pallas import tpu_sc as plsc

NC = 2
NS = 16
NW = NC * NS
LANES = 16
GRID_R = 64


@functools.partial(jax.jit, static_argnums=(9, 10, 11))
def _sc_loss(e0, e1, refx, refy, refz, gridf, vxh, vyh, vzh, N, C, CV):
    E = e0.shape[0]
    EPT = E // NW
    NCH = EPT // C

    mesh = plsc.VectorSubcoreMesh(
        core_axis_name="c", subcore_axis_name="s",
        num_cores=NC, num_subcores=NS)

    scratch = [
        pltpu.VMEM((C,), jnp.int32),
        pltpu.VMEM((C,), jnp.int32),
    ]
    scratch += [pltpu.VMEM((C,), jnp.float32) for _ in range(6)]
    scratch += [pltpu.VMEM((C,), jnp.float32) for _ in range(3)]
    scratch += [
        pltpu.VMEM((CV,), jnp.float32),
        pltpu.VMEM((CV,), jnp.float32),
        pltpu.VMEM((CV,), jnp.float32),
        pltpu.VMEM((CV,), jnp.float32),
        pltpu.VMEM((CV,), jnp.float32),
        pltpu.VMEM((CV,), jnp.float32),
    ]
    scratch += [pltpu.VMEM((CV,), jnp.int32) for _ in range(8)]
    scratch += [pltpu.VMEM((CV,), jnp.float32) for _ in range(8)]
    scratch += [
        pltpu.VMEM((LANES,), jnp.float32),
        pltpu.SemaphoreType.DMA,
        pltpu.SemaphoreType.DMA,
    ]

    @functools.partial(
        pl.kernel,
        out_type=jax.ShapeDtypeStruct((NW, LANES), jnp.float32),
        mesh=mesh,
        scratch_types=scratch,
        compiler_params=pltpu.CompilerParams(
            needs_layout_passes=False, use_tc_tiling_on_sc=False),
    )
    def k(e0_h, e1_h, rx_h, ry_h, rz_h, gridf_h, vx_h, vy_h, vz_h, out_h, *scr):
        idx0_v, idx1_v = scr[0:2]
        gx0_v, gy0_v, gz0_v, gx1_v, gy1_v, gz1_v = scr[2:8]
        rx_v, ry_v, rz_v = scr[8:11]
        vx_v, vy_v, vz_v, fx_v, fy_v, fz_v = scr[11:17]
        cidx = scr[17:25]
        cval = scr[25:33]
        acc_v = scr[33]
        sem0, sem1 = scr[34:36]

        wid = lax.axis_index("s") * NC + lax.axis_index("c")
        iot = lax.iota(jnp.int32, LANES)

        eb = wid * EPT

        def echunk(i, acc):
            base = eb + i * C
            sl = pl.ds(base, C)
            cpr0 = pltpu.async_copy(rx_h.at[sl], rx_v, sem1)
            cpr1 = pltpu.async_copy(ry_h.at[sl], ry_v, sem1)
            cpr2 = pltpu.async_copy(rz_h.at[sl], rz_v, sem1)
            pltpu.sync_copy(e0_h.at[sl], idx0_v)
            pltpu.sync_copy(e1_h.at[sl], idx1_v)
            cps = [
                pltpu.async_copy(vx_h.at[idx0_v], gx0_v, sem0),
                pltpu.async_copy(vy_h.at[idx0_v], gy0_v, sem0),
                pltpu.async_copy(vz_h.at[idx0_v], gz0_v, sem0),
                pltpu.async_copy(vx_h.at[idx1_v], gx1_v, sem0),
                pltpu.async_copy(vy_h.at[idx1_v], gy1_v, sem0),
                pltpu.async_copy(vz_h.at[idx1_v], gz1_v, sem0),
            ]
            for cp in (cpr0, cpr1, cpr2, *cps):
                cp.wait()

            def grp(g, a):
                s = pl.ds(g * LANES, LANES)
                dx = gx0_v[s] - gx1_v[s] - rx_v[s]
                dy = gy0_v[s] - gy1_v[s] - ry_v[s]
                dz = gz0_v[s] - gz1_v[s] - rz_v[s]
                return a + (dx * dx + dy * dy + dz * dz)

            return lax.fori_loop(0, C // LANES, grp, acc)

        acc = lax.fori_loop(0, NCH, echunk, jnp.zeros((LANES,), jnp.float32))

        vb = wid * CV
        pltpu.sync_copy(vx_h.at[pl.ds(vb, CV)], vx_v)
        pltpu.sync_copy(vy_h.at[pl.ds(vb, CV)], vy_v)
        pltpu.sync_copy(vz_h.at[pl.ds(vb, CV)], vz_v)

        def vprep(g, carry):
            sl = pl.ds(g * LANES, LANES)

            def prep(p):
                u = jnp.minimum(
                    jnp.maximum((p + 1.0) * 0.5 * float(GRID_R - 1), 0.0),
                    float(GRID_R - 1) - 1e-4)
                i0 = u.astype(jnp.int32)
                return i0, u - i0.astype(jnp.float32)

            x0, fx = prep(vx_v[sl])
            y0, fy = prep(vy_v[sl])
            z0, fz = prep(vz_v[sl])
            b = x0 * (GRID_R * GRID_R) + y0 * GRID_R + z0
            cidx[0][sl] = b
            cidx[1][sl] = b + 1
            cidx[2][sl] = b + GRID_R
            cidx[3][sl] = b + GRID_R + 1
            cidx[4][sl] = b + GRID_R * GRID_R
            cidx[5][sl] = b + GRID_R * GRID_R + 1
            cidx[6][sl] = b + GRID_R * GRID_R + GRID_R
            cidx[7][sl] = b + GRID_R * GRID_R + GRID_R + 1
            fx_v[sl] = fx
            fy_v[sl] = fy
            fz_v[sl] = fz
            return carry

        lax.fori_loop(0, CV // LANES, vprep, 0)

        cps = [pltpu.async_copy(gridf_h.at[cidx[j]], cval[j], sem0)
               for j in range(8)]
        for cp in cps:
            cp.wait()

        def vgrp(g, a):
            sl = pl.ds(g * LANES, LANES)
            fx = fx_v[sl]
            fy = fy_v[sl]
            fz = fz_v[sl]
            c00 = cval[0][sl] * (1 - fx) + cval[4][sl] * fx
            c10 = cval[2][sl] * (1 - fx) + cval[6][sl] * fx
            c01 = cval[1][sl] * (1 - fx) + cval[5][sl] * fx
            c11 = cval[3][sl] * (1 - fx) + cval[7][sl] * fx
            c0 = c00 * (1 - fy) + c10 * fy
            c1 = c01 * (1 - fy) + c11 * fy
            d = c0 * (1 - fz) + c1 * fz
            vid = vb + g * LANES + iot
            dm = jnp.where(vid < N, d, 0.0)
            return a + dm * dm

        acc = lax.fori_loop(0, CV // LANES, vgrp, acc)

        acc_v[...] = acc
        pltpu.sync_copy(acc_v, out_h.at[wid])

    return k(e0, e1, refx, refy, refz, gridf, vxh, vyh, vzh)


def kernel(src_V, src_E, dist_grid, ref_edge_vec):
    N = src_V.shape[0]
    E = src_E.shape[0]
    assert E % NW == 0
    ept = E // NW
    C = 16
    for cand in range(16, 4097, 16):
        if ept % cand == 0:
            C = cand
    align = LANES * NW
    NP = ((N + align - 1) // align) * align
    CV = NP // NW

    e0 = src_E[:, 0]
    e1 = src_E[:, 1]
    refx = ref_edge_vec[:, 0]
    refy = ref_edge_vec[:, 1]
    refz = ref_edge_vec[:, 2]
    gridf = dist_grid.reshape(-1)
    pad = NP - N
    vx = jnp.pad(src_V[:, 0], (0, pad))
    vy = jnp.pad(src_V[:, 1], (0, pad))
    vz = jnp.pad(src_V[:, 2], (0, pad))

    out = _sc_loss(e0, e1, refx, refy, refz, gridf, vx, vy, vz, N, C, CV)
    return 0.5 * jnp.sum(out)


# speedup vs baseline: 12.2901x; 1.1022x over previous
import functools

import jax
import jax.numpy as jnp
from jax import lax
from jax.experimental import pallas as pl
from jax.experimental.pallas import tpu as pltpu
from jax.experimental.pallas import tpu_sc as plsc

NC = 2
NS = 16
NW = NC * NS
LANES = 16
GRID_R = 64


@functools.partial(jax.jit, static_argnums=(9, 10, 11))
def _sc_loss(e0, e1, refx, refy, refz, gridf, vxh, vyh, vzh, N, C, CV):
    E = e0.shape[0]
    EPT = E // NW
    NCH = EPT // C
    NPAIR = (NCH - 1) // 2

    mesh = plsc.VectorSubcoreMesh(
        core_axis_name="c", subcore_axis_name="s",
        num_cores=NC, num_subcores=NS)

    def edge_set():
        return (
            [pltpu.VMEM((C,), jnp.int32) for _ in range(2)]
            + [pltpu.VMEM((C,), jnp.float32) for _ in range(6)]
            + [pltpu.VMEM((C,), jnp.float32) for _ in range(3)]
        )

    scratch = edge_set() + edge_set()
    scratch += [pltpu.VMEM((CV,), jnp.float32) for _ in range(6)]
    scratch += [pltpu.VMEM((CV,), jnp.int32) for _ in range(8)]
    scratch += [pltpu.VMEM((CV,), jnp.float32) for _ in range(8)]
    scratch += [
        pltpu.VMEM((LANES,), jnp.float32),
        pltpu.SemaphoreType.DMA,
        pltpu.SemaphoreType.DMA,
        pltpu.SemaphoreType.DMA,
    ]

    @functools.partial(
        pl.kernel,
        out_type=jax.ShapeDtypeStruct((NW, LANES), jnp.float32),
        mesh=mesh,
        scratch_types=scratch,
        compiler_params=pltpu.CompilerParams(
            needs_layout_passes=False, use_tc_tiling_on_sc=False),
    )
    def k(e0_h, e1_h, rx_h, ry_h, rz_h, gridf_h, vx_h, vy_h, vz_h, out_h, *scr):
        bufA = scr[0:11]
        bufB = scr[11:22]
        vx_v, vy_v, vz_v, fx_v, fy_v, fz_v = scr[22:28]
        cidx = scr[28:36]
        cval = scr[36:44]
        acc_v = scr[44]
        semG, semI, semC = scr[45:48]

        wid = lax.axis_index("s") * NC + lax.axis_index("c")
        iot = lax.iota(jnp.int32, LANES)
        eb = wid * EPT

        def fire_idx(buf, base):
            pltpu.async_copy(e0_h.at[pl.ds(base, C)], buf[0], semI)
            pltpu.async_copy(e1_h.at[pl.ds(base, C)], buf[1], semI)

        def wait_idx(buf):
            pltpu.make_async_copy(e0_h.at[pl.ds(0, C)], buf[0], semI).wait()
            pltpu.make_async_copy(e1_h.at[pl.ds(0, C)], buf[1], semI).wait()

        def fire_gathers(buf, base):
            idx0_v, idx1_v = buf[0], buf[1]
            pltpu.async_copy(vx_h.at[idx0_v], buf[2], semG)
            pltpu.async_copy(vy_h.at[idx0_v], buf[3], semG)
            pltpu.async_copy(vz_h.at[idx0_v], buf[4], semG)
            pltpu.async_copy(vx_h.at[idx1_v], buf[5], semG)
            pltpu.async_copy(vy_h.at[idx1_v], buf[6], semG)
            pltpu.async_copy(vz_h.at[idx1_v], buf[7], semG)
            sl = pl.ds(base, C)
            pltpu.async_copy(rx_h.at[sl], buf[8], semG)
            pltpu.async_copy(ry_h.at[sl], buf[9], semG)
            pltpu.async_copy(rz_h.at[sl], buf[10], semG)

        def wait_gathers(buf):
            for j in range(2, 8):
                pltpu.make_async_copy(vx_h.at[buf[0]], buf[j], semG).wait()
            for j in range(8, 11):
                pltpu.make_async_copy(rx_h.at[pl.ds(0, C)], buf[j], semG).wait()

        def compute(buf, acc):
            def grp(g, a):
                s = pl.ds(g * LANES, LANES)
                dx = buf[2][s] - buf[5][s] - buf[8][s]
                dy = buf[3][s] - buf[6][s] - buf[9][s]
                dz = buf[4][s] - buf[7][s] - buf[10][s]
                return a + (dx * dx + dy * dy + dz * dz)

            return lax.fori_loop(0, C // LANES, grp, acc)

        fire_idx(bufA, eb)

        vb = wid * CV
        pltpu.sync_copy(vx_h.at[pl.ds(vb, CV)], vx_v)
        pltpu.sync_copy(vy_h.at[pl.ds(vb, CV)], vy_v)
        pltpu.sync_copy(vz_h.at[pl.ds(vb, CV)], vz_v)

        def vprep(g, carry):
            sl = pl.ds(g * LANES, LANES)

            def prep(p):
                u = jnp.minimum(
                    jnp.maximum((p + 1.0) * 0.5 * float(GRID_R - 1), 0.0),
                    float(GRID_R - 1) - 1e-4)
                i0 = u.astype(jnp.int32)
                return i0, u - i0.astype(jnp.float32)

            x0, fx = prep(vx_v[sl])
            y0, fy = prep(vy_v[sl])
            z0, fz = prep(vz_v[sl])
            b = x0 * (GRID_R * GRID_R) + y0 * GRID_R + z0
            cidx[0][sl] = b
            cidx[1][sl] = b + 1
            cidx[2][sl] = b + GRID_R
            cidx[3][sl] = b + GRID_R + 1
            cidx[4][sl] = b + GRID_R * GRID_R
            cidx[5][sl] = b + GRID_R * GRID_R + 1
            cidx[6][sl] = b + GRID_R * GRID_R + GRID_R
            cidx[7][sl] = b + GRID_R * GRID_R + GRID_R + 1
            fx_v[sl] = fx
            fy_v[sl] = fy
            fz_v[sl] = fz
            return carry

        lax.fori_loop(0, CV // LANES, vprep, 0)

        for j in range(8):
            pltpu.async_copy(gridf_h.at[cidx[j]], cval[j], semC)

        wait_idx(bufA)
        fire_gathers(bufA, eb)
        wait_gathers(bufA)
        acc = compute(bufA, jnp.zeros((LANES,), jnp.float32))
        if NCH > 1:
            fire_idx(bufA, eb + C)
            wait_idx(bufA)
            fire_gathers(bufA, eb + C)

        def pair(j, acc):
            baseA = eb + (2 * j + 1) * C
            baseB = baseA + C
            fire_idx(bufB, baseB)
            wait_idx(bufB)
            fire_gathers(bufB, baseB)
            wait_gathers(bufA)
            acc = compute(bufA, acc)

            @pl.when(j + 1 < NPAIR)
            def _():
                fire_idx(bufA, baseB + C)
                wait_idx(bufA)
                fire_gathers(bufA, baseB + C)

            wait_gathers(bufB)
            return compute(bufB, acc)

        acc = lax.fori_loop(0, NPAIR, pair, acc)

        for j in range(8):
            pltpu.make_async_copy(gridf_h.at[cidx[j]], cval[j], semC).wait()

        def vgrp(g, a):
            sl = pl.ds(g * LANES, LANES)
            fx = fx_v[sl]
            fy = fy_v[sl]
            fz = fz_v[sl]
            c00 = cval[0][sl] * (1 - fx) + cval[4][sl] * fx
            c10 = cval[2][sl] * (1 - fx) + cval[6][sl] * fx
            c01 = cval[1][sl] * (1 - fx) + cval[5][sl] * fx
            c11 = cval[3][sl] * (1 - fx) + cval[7][sl] * fx
            c0 = c00 * (1 - fy) + c10 * fy
            c1 = c01 * (1 - fy) + c11 * fy
            d = c0 * (1 - fz) + c1 * fz
            vid = vb + g * LANES + iot
            dm = jnp.where(vid < N, d, 0.0)
            return a + dm * dm

        acc = lax.fori_loop(0, CV // LANES, vgrp, acc)

        acc_v[...] = acc
        pltpu.sync_copy(acc_v, out_h.at[wid])

    return k(e0, e1, refx, refy, refz, gridf, vxh, vyh, vzh)


def kernel(src_V, src_E, dist_grid, ref_edge_vec):
    N = src_V.shape[0]
    E = src_E.shape[0]
    assert E % NW == 0
    ept = E // NW
    C = 16
    for cand in range(16, 2049, 16):
        if ept % cand == 0:
            C = cand
    align = LANES * NW
    NP = ((N + align - 1) // align) * align
    CV = NP // NW

    e0 = src_E[:, 0]
    e1 = src_E[:, 1]
    refx = ref_edge_vec[:, 0]
    refy = ref_edge_vec[:, 1]
    refz = ref_edge_vec[:, 2]
    gridf = dist_grid.reshape(-1)
    pad = NP - N
    vx = jnp.pad(src_V[:, 0], (0, pad))
    vy = jnp.pad(src_V[:, 1], (0, pad))
    vz = jnp.pad(src_V[:, 2], (0, pad))

    out = _sc_loss(e0, e1, refx, refy, refz, gridf, vx, vy, vz, N, C, CV)
    return 0.5 * jnp.sum(out)


# speedup vs baseline: 15.6030x; 1.2696x over previous
import functools

import jax
import jax.numpy as jnp
from jax import lax
from jax.experimental import pallas as pl
from jax.experimental.pallas import tpu as pltpu
from jax.experimental.pallas import tpu_sc as plsc

NC = 2
NS = 16
NW = NC * NS
LANES = 16
GRID_R = 64


@functools.partial(jax.jit, static_argnums=(9, 10, 11))
def _sc_loss(e0, e1, refx, refy, refz, gridf, vxh, vyh, vzh, N, C, CV):
    E = e0.shape[0]
    EPT = E // NW
    NCH = EPT // C
    NPAIR = (NCH - 1) // 2
    NP = CV * NW
    RPT = NP // NS
    BLK = RPT // 8
    NBLK = RPT // BLK

    mesh = plsc.VectorSubcoreMesh(
        core_axis_name="c", subcore_axis_name="s",
        num_cores=NC, num_subcores=NS)

    def edge_set():
        return (
            [pltpu.VMEM((C,), jnp.int32) for _ in range(2)]
            + [pltpu.VMEM((C, LANES), jnp.float32) for _ in range(2)]
            + [pltpu.VMEM((C,), jnp.float32) for _ in range(3)]
        )

    scratch = edge_set() + edge_set()
    scratch += [pltpu.VMEM((BLK,), jnp.float32) for _ in range(3)]
    scratch += [pltpu.VMEM((BLK, LANES), jnp.float32)]
    scratch += [pltpu.VMEM((CV,), jnp.float32) for _ in range(6)]
    scratch += [pltpu.VMEM((CV,), jnp.int32) for _ in range(8)]
    scratch += [pltpu.VMEM((CV,), jnp.float32) for _ in range(8)]
    scratch += [
        pltpu.VMEM((LANES,), jnp.float32),
        pltpu.SemaphoreType.DMA,
        pltpu.SemaphoreType.DMA,
        pltpu.SemaphoreType.DMA,
    ]

    @functools.partial(
        pl.kernel,
        out_type=(jax.ShapeDtypeStruct((NW, LANES), jnp.float32),
                  jax.ShapeDtypeStruct((NP, LANES), jnp.float32)),
        mesh=mesh,
        scratch_types=scratch,
        compiler_params=pltpu.CompilerParams(
            needs_layout_passes=False, use_tc_tiling_on_sc=False),
    )
    def k(e0_h, e1_h, rx_h, ry_h, rz_h, gridf_h, vx_h, vy_h, vz_h,
          out_h, tab_h, *scr):
        bufA = scr[0:7]
        bufB = scr[7:14]
        bvx, bvy, bvz = scr[14:17]
        stag = scr[17]
        vx_v, vy_v, vz_v, fx_v, fy_v, fz_v = scr[18:24]
        cidx = scr[24:32]
        cval = scr[32:40]
        acc_v = scr[40]
        semG, semI, semC = scr[41:44]

        sid = lax.axis_index("s")
        wid = sid * NC + lax.axis_index("c")
        iot = lax.iota(jnp.int32, LANES)
        colx = iot * 0
        coly = colx + 1
        colz = colx + 2
        eb = wid * EPT

        tbase = sid * RPT

        def build_blk(b, carry):
            rb = tbase + b * BLK
            pltpu.sync_copy(vx_h.at[pl.ds(rb, BLK)], bvx)
            pltpu.sync_copy(vy_h.at[pl.ds(rb, BLK)], bvy)
            pltpu.sync_copy(vz_h.at[pl.ds(rb, BLK)], bvz)

            def grp(g, c):
                sl = pl.ds(g * LANES, LANES)
                srow = g * LANES + iot
                plsc.store_scatter(stag, [srow, colx], bvx[sl])
                plsc.store_scatter(stag, [srow, coly], bvy[sl])
                plsc.store_scatter(stag, [srow, colz], bvz[sl])
                return c

            lax.fori_loop(0, BLK // LANES, grp, 0)
            pltpu.sync_copy(stag, tab_h.at[pl.ds(rb, BLK)])
            return carry

        lax.fori_loop(0, NBLK, build_blk, 0)
        plsc.subcore_barrier()

        def fire_idx(buf, base):
            pltpu.async_copy(e0_h.at[pl.ds(base, C)], buf[0], semI)
            pltpu.async_copy(e1_h.at[pl.ds(base, C)], buf[1], semI)

        def wait_idx(buf):
            pltpu.make_async_copy(e0_h.at[pl.ds(0, C)], buf[0], semI).wait()
            pltpu.make_async_copy(e1_h.at[pl.ds(0, C)], buf[1], semI).wait()

        def fire_gathers(buf, base):
            pltpu.async_copy(tab_h.at[buf[0]], buf[2], semG)
            pltpu.async_copy(tab_h.at[buf[1]], buf[3], semG)
            sl = pl.ds(base, C)
            pltpu.async_copy(rx_h.at[sl], buf[4], semG)
            pltpu.async_copy(ry_h.at[sl], buf[5], semG)
            pltpu.async_copy(rz_h.at[sl], buf[6], semG)

        def wait_gathers(buf):
            for j in (2, 3):
                pltpu.make_async_copy(tab_h.at[buf[0]], buf[j], semG).wait()
            for j in (4, 5, 6):
                pltpu.make_async_copy(rx_h.at[pl.ds(0, C)], buf[j], semG).wait()

        def compute(buf, acc):
            r0, r1 = buf[2], buf[3]

            def grp(g, a):
                s = pl.ds(g * LANES, LANES)
                row = g * LANES + iot
                dx = (plsc.load_gather(r0, [row, colx])
                      - plsc.load_gather(r1, [row, colx]) - buf[4][s])
                dy = (plsc.load_gather(r0, [row, coly])
                      - plsc.load_gather(r1, [row, coly]) - buf[5][s])
                dz = (plsc.load_gather(r0, [row, colz])
                      - plsc.load_gather(r1, [row, colz]) - buf[6][s])
                return a + (dx * dx + dy * dy + dz * dz)

            return lax.fori_loop(0, C // LANES, grp, acc)

        fire_idx(bufA, eb)

        vb = wid * CV
        pltpu.sync_copy(vx_h.at[pl.ds(vb, CV)], vx_v)
        pltpu.sync_copy(vy_h.at[pl.ds(vb, CV)], vy_v)
        pltpu.sync_copy(vz_h.at[pl.ds(vb, CV)], vz_v)

        def vprep(g, carry):
            sl = pl.ds(g * LANES, LANES)

            def prep(p):
                u = jnp.minimum(
                    jnp.maximum((p + 1.0) * 0.5 * float(GRID_R - 1), 0.0),
                    float(GRID_R - 1) - 1e-4)
                i0 = u.astype(jnp.int32)
                return i0, u - i0.astype(jnp.float32)

            x0, fx = prep(vx_v[sl])
            y0, fy = prep(vy_v[sl])
            z0, fz = prep(vz_v[sl])
            b = x0 * (GRID_R * GRID_R) + y0 * GRID_R + z0
            cidx[0][sl] = b
            cidx[1][sl] = b + 1
            cidx[2][sl] = b + GRID_R
            cidx[3][sl] = b + GRID_R + 1
            cidx[4][sl] = b + GRID_R * GRID_R
            cidx[5][sl] = b + GRID_R * GRID_R + 1
            cidx[6][sl] = b + GRID_R * GRID_R + GRID_R
            cidx[7][sl] = b + GRID_R * GRID_R + GRID_R + 1
            fx_v[sl] = fx
            fy_v[sl] = fy
            fz_v[sl] = fz
            return carry

        lax.fori_loop(0, CV // LANES, vprep, 0)

        for j in range(8):
            pltpu.async_copy(gridf_h.at[cidx[j]], cval[j], semC)

        wait_idx(bufA)
        fire_gathers(bufA, eb)
        wait_gathers(bufA)
        acc = compute(bufA, jnp.zeros((LANES,), jnp.float32))
        if NCH > 1:
            fire_idx(bufA, eb + C)
            wait_idx(bufA)
            fire_gathers(bufA, eb + C)

        def pair(j, acc):
            baseA = eb + (2 * j + 1) * C
            baseB = baseA + C
            fire_idx(bufB, baseB)
            wait_idx(bufB)
            fire_gathers(bufB, baseB)
            wait_gathers(bufA)
            acc = compute(bufA, acc)

            @pl.when(j + 1 < NPAIR)
            def _():
                fire_idx(bufA, baseB + C)
                wait_idx(bufA)
                fire_gathers(bufA, baseB + C)

            wait_gathers(bufB)
            return compute(bufB, acc)

        acc = lax.fori_loop(0, NPAIR, pair, acc)

        for j in range(8):
            pltpu.make_async_copy(gridf_h.at[cidx[j]], cval[j], semC).wait()

        def vgrp(g, a):
            sl = pl.ds(g * LANES, LANES)
            fx = fx_v[sl]
            fy = fy_v[sl]
            fz = fz_v[sl]
            c00 = cval[0][sl] * (1 - fx) + cval[4][sl] * fx
            c10 = cval[2][sl] * (1 - fx) + cval[6][sl] * fx
            c01 = cval[1][sl] * (1 - fx) + cval[5][sl] * fx
            c11 = cval[3][sl] * (1 - fx) + cval[7][sl] * fx
            c0 = c00 * (1 - fy) + c10 * fy
            c1 = c01 * (1 - fy) + c11 * fy
            d = c0 * (1 - fz) + c1 * fz
            vid = vb + g * LANES + iot
            dm = jnp.where(vid < N, d, 0.0)
            return a + dm * dm

        acc = lax.fori_loop(0, CV // LANES, vgrp, acc)

        acc_v[...] = acc
        pltpu.sync_copy(acc_v, out_h.at[wid])

    return k(e0, e1, refx, refy, refz, gridf, vxh, vyh, vzh)[0]


def kernel(src_V, src_E, dist_grid, ref_edge_vec):
    N = src_V.shape[0]
    E = src_E.shape[0]
    assert E % NW == 0
    ept = E // NW
    C = 16
    for cand in range(16, 1025, 16):
        if ept % cand == 0 and (ept // cand) % 2 == 1:
            C = cand
    align = LANES * NW
    NP = ((N + align - 1) // align) * align
    CV = NP // NW

    e0 = src_E[:, 0]
    e1 = src_E[:, 1]
    refx = ref_edge_vec[:, 0]
    refy = ref_edge_vec[:, 1]
    refz = ref_edge_vec[:, 2]
    gridf = dist_grid.reshape(-1)
    pad = NP - N
    vx = jnp.pad(src_V[:, 0], (0, pad))
    vy = jnp.pad(src_V[:, 1], (0, pad))
    vz = jnp.pad(src_V[:, 2], (0, pad))

    out = _sc_loss(e0, e1, refx, refy, refz, gridf, vx, vy, vz, N, C, CV)
    return 0.5 * jnp.sum(out)


# speedup vs baseline: 16.3525x; 1.0480x over previous
import functools

import jax
import jax.numpy as jnp
from jax import lax
from jax.experimental import pallas as pl
from jax.experimental.pallas import tpu as pltpu
from jax.experimental.pallas import tpu_sc as plsc

NC = 2
NS = 16
NW = NC * NS
LANES = 16
GRID_R = 64


@functools.partial(jax.jit, static_argnums=(9, 10, 11))
def _sc_loss(e0, e1, refx, refy, refz, gridf, vxh, vyh, vzh, N, C, CV):
    E = e0.shape[0]
    EPT = E // NW
    NCH = EPT // C
    NPAIR = (NCH - 1) // 2
    NP = CV * NW
    RPT = NP // NS
    BLK = 224
    assert RPT % BLK == 0
    NBLK = RPT // BLK

    mesh = plsc.VectorSubcoreMesh(
        core_axis_name="c", subcore_axis_name="s",
        num_cores=NC, num_subcores=NS)

    W = 8

    def edge_set():
        return (
            [pltpu.VMEM((C,), jnp.int32) for _ in range(2)]
            + [pltpu.VMEM((C, W), jnp.float32) for _ in range(2)]
            + [pltpu.VMEM((C,), jnp.float32) for _ in range(3)]
        )

    scratch = edge_set() + edge_set()
    scratch += [pltpu.VMEM((BLK,), jnp.float32) for _ in range(3)]
    scratch += [pltpu.VMEM((BLK, W), jnp.float32)]
    H = CV // 2
    scratch += [pltpu.VMEM((H,), jnp.float32) for _ in range(6)]
    scratch += [pltpu.VMEM((H,), jnp.int32) for _ in range(8)]
    scratch += [pltpu.VMEM((H,), jnp.float32) for _ in range(8)]
    scratch += [
        pltpu.VMEM((LANES,), jnp.float32),
        pltpu.SemaphoreType.DMA,
        pltpu.SemaphoreType.DMA,
        pltpu.SemaphoreType.DMA,
    ]

    @functools.partial(
        pl.kernel,
        out_type=(jax.ShapeDtypeStruct((NW, LANES), jnp.float32),
                  jax.ShapeDtypeStruct((NP, W), jnp.float32)),
        mesh=mesh,
        scratch_types=scratch,
        compiler_params=pltpu.CompilerParams(
            needs_layout_passes=False, use_tc_tiling_on_sc=False),
    )
    def k(e0_h, e1_h, rx_h, ry_h, rz_h, gridf_h, vx_h, vy_h, vz_h,
          out_h, tab_h, *scr):
        bufA = scr[0:7]
        bufB = scr[7:14]
        bvx, bvy, bvz = scr[14:17]
        stag = scr[17]
        vh = scr[18:24]
        cidx = scr[24:32]
        cval = scr[32:40]
        acc_v = scr[40]
        semG, semI, semC = scr[41:44]

        sid = lax.axis_index("s")
        wid = sid * NC + lax.axis_index("c")
        iot = lax.iota(jnp.int32, LANES)
        colx = iot * 0
        coly = colx + 1
        colz = colx + 2
        eb = wid * EPT

        tbase = sid * RPT

        def build_blk(b, carry):
            rb = tbase + b * BLK
            cb0 = pltpu.async_copy(vx_h.at[pl.ds(rb, BLK)], bvx, semI)
            cb1 = pltpu.async_copy(vy_h.at[pl.ds(rb, BLK)], bvy, semI)
            cb2 = pltpu.async_copy(vz_h.at[pl.ds(rb, BLK)], bvz, semI)
            cb0.wait()
            cb1.wait()
            cb2.wait()

            def grp(g, c):
                sl = pl.ds(g * LANES, LANES)
                srow = g * LANES + iot
                plsc.store_scatter(stag, [srow, colx], bvx[sl])
                plsc.store_scatter(stag, [srow, coly], bvy[sl])
                plsc.store_scatter(stag, [srow, colz], bvz[sl])
                return c

            lax.fori_loop(0, BLK // LANES, grp, 0)
            pltpu.sync_copy(stag, tab_h.at[pl.ds(rb, BLK)])
            return carry

        lax.fori_loop(0, NBLK, build_blk, 0)
        plsc.subcore_barrier()

        def fire_idx(buf, base):
            pltpu.async_copy(e0_h.at[pl.ds(base, C)], buf[0], semI)
            pltpu.async_copy(e1_h.at[pl.ds(base, C)], buf[1], semI)

        def wait_idx(buf):
            pltpu.make_async_copy(e0_h.at[pl.ds(0, C)], buf[0], semI).wait()
            pltpu.make_async_copy(e1_h.at[pl.ds(0, C)], buf[1], semI).wait()

        def fire_gathers(buf, base):
            pltpu.async_copy(tab_h.at[buf[0]], buf[2], semG)
            pltpu.async_copy(tab_h.at[buf[1]], buf[3], semG)
            sl = pl.ds(base, C)
            pltpu.async_copy(rx_h.at[sl], buf[4], semG)
            pltpu.async_copy(ry_h.at[sl], buf[5], semG)
            pltpu.async_copy(rz_h.at[sl], buf[6], semG)

        def wait_gathers(buf):
            for j in (2, 3):
                pltpu.make_async_copy(tab_h.at[buf[0]], buf[j], semG).wait()
            for j in (4, 5, 6):
                pltpu.make_async_copy(rx_h.at[pl.ds(0, C)], buf[j], semG).wait()

        def compute(buf, acc):
            r0, r1 = buf[2], buf[3]

            def grp(g, a):
                s = pl.ds(g * LANES, LANES)
                row = g * LANES + iot
                dx = (plsc.load_gather(r0, [row, colx])
                      - plsc.load_gather(r1, [row, colx]) - buf[4][s])
                dy = (plsc.load_gather(r0, [row, coly])
                      - plsc.load_gather(r1, [row, coly]) - buf[5][s])
                dz = (plsc.load_gather(r0, [row, colz])
                      - plsc.load_gather(r1, [row, colz]) - buf[6][s])
                return a + (dx * dx + dy * dy + dz * dz)

            return lax.fori_loop(0, C // LANES, grp, acc)

        fire_idx(bufA, eb)

        vb = wid * CV
        cvl = [
            pltpu.async_copy(vx_h.at[pl.ds(vb, H)], vh[0], semC),
            pltpu.async_copy(vy_h.at[pl.ds(vb, H)], vh[1], semC),
            pltpu.async_copy(vz_h.at[pl.ds(vb, H)], vh[2], semC),
            pltpu.async_copy(vx_h.at[pl.ds(vb + H, H)], vh[3], semC),
            pltpu.async_copy(vy_h.at[pl.ds(vb + H, H)], vh[4], semC),
            pltpu.async_copy(vz_h.at[pl.ds(vb + H, H)], vh[5], semC),
        ]
        for cp in cvl:
            cp.wait()

        def prep(p):
            u = jnp.minimum(
                jnp.maximum((p + 1.0) * 0.5 * float(GRID_R - 1), 0.0),
                float(GRID_R - 1) - 1e-4)
            i0 = u.astype(jnp.int32)
            return i0, u - i0.astype(jnp.float32)

        def vprep(half, g, carry):
            sl = pl.ds(g * LANES, LANES)
            x0, fx = prep(vh[3 * half + 0][sl])
            y0, fy = prep(vh[3 * half + 1][sl])
            z0, fz = prep(vh[3 * half + 2][sl])
            b = x0 * (GRID_R * GRID_R) + y0 * GRID_R + z0
            cidx[0][sl] = b
            cidx[1][sl] = b + 1
            cidx[2][sl] = b + GRID_R
            cidx[3][sl] = b + GRID_R + 1
            cidx[4][sl] = b + GRID_R * GRID_R
            cidx[5][sl] = b + GRID_R * GRID_R + 1
            cidx[6][sl] = b + GRID_R * GRID_R + GRID_R
            cidx[7][sl] = b + GRID_R * GRID_R + GRID_R + 1
            return carry

        lax.fori_loop(0, H // LANES, functools.partial(vprep, 0), 0)

        for j in range(8):
            pltpu.async_copy(gridf_h.at[cidx[j]], cval[j], semC)

        wait_idx(bufA)
        fire_gathers(bufA, eb)
        wait_gathers(bufA)
        acc = compute(bufA, jnp.zeros((LANES,), jnp.float32))
        if NCH > 1:
            fire_idx(bufA, eb + C)
            wait_idx(bufA)
            fire_gathers(bufA, eb + C)

        def pair(j, acc):
            baseA = eb + (2 * j + 1) * C
            baseB = baseA + C
            fire_idx(bufB, baseB)
            wait_idx(bufB)
            fire_gathers(bufB, baseB)
            wait_gathers(bufA)
            acc = compute(bufA, acc)

            @pl.when(j + 1 < NPAIR)
            def _():
                fire_idx(bufA, baseB + C)
                wait_idx(bufA)
                fire_gathers(bufA, baseB + C)

            wait_gathers(bufB)
            return compute(bufB, acc)

        acc = lax.fori_loop(0, NPAIR, pair, acc)

        def vgrp(half, g, a):
            sl = pl.ds(g * LANES, LANES)
            _, fx = prep(vh[3 * half + 0][sl])
            _, fy = prep(vh[3 * half + 1][sl])
            _, fz = prep(vh[3 * half + 2][sl])
            c00 = cval[0][sl] * (1 - fx) + cval[4][sl] * fx
            c10 = cval[2][sl] * (1 - fx) + cval[6][sl] * fx
            c01 = cval[1][sl] * (1 - fx) + cval[5][sl] * fx
            c11 = cval[3][sl] * (1 - fx) + cval[7][sl] * fx
            c0 = c00 * (1 - fy) + c10 * fy
            c1 = c01 * (1 - fy) + c11 * fy
            d = c0 * (1 - fz) + c1 * fz
            vid = vb + half * H + g * LANES + iot
            dm = jnp.where(vid < N, d, 0.0)
            return a + dm * dm

        for j in range(8):
            pltpu.make_async_copy(gridf_h.at[cidx[j]], cval[j], semC).wait()
        acc = lax.fori_loop(0, H // LANES, functools.partial(vgrp, 0), acc)

        lax.fori_loop(0, H // LANES, functools.partial(vprep, 1), 0)
        cps2 = [pltpu.async_copy(gridf_h.at[cidx[j]], cval[j], semC)
                for j in range(8)]
        for cp in cps2:
            cp.wait()
        acc = lax.fori_loop(0, H // LANES, functools.partial(vgrp, 1), acc)

        acc_v[...] = acc
        pltpu.sync_copy(acc_v, out_h.at[wid])

    return k(e0, e1, refx, refy, refz, gridf, vxh, vyh, vzh)[0]


def kernel(src_V, src_E, dist_grid, ref_edge_vec):
    N = src_V.shape[0]
    E = src_E.shape[0]
    assert E % NW == 0
    ept = E // NW
    C = 16
    for cand in range(16, 1025, 16):
        if ept % cand == 0 and (ept // cand) % 2 == 1:
            C = cand
    align = LANES * NW
    NP = ((N + align - 1) // align) * align
    CV = NP // NW

    e0 = src_E[:, 0]
    e1 = src_E[:, 1]
    refx = ref_edge_vec[:, 0]
    refy = ref_edge_vec[:, 1]
    refz = ref_edge_vec[:, 2]
    gridf = dist_grid.reshape(-1)
    pad = NP - N
    vx = jnp.pad(src_V[:, 0], (0, pad))
    vy = jnp.pad(src_V[:, 1], (0, pad))
    vz = jnp.pad(src_V[:, 2], (0, pad))

    out = _sc_loss(e0, e1, refx, refy, refz, gridf, vx, vy, vz, N, C, CV)
    return 0.5 * jnp.sum(out)


# speedup vs baseline: 19.9312x; 1.2188x over previous
import functools

import jax
import jax.numpy as jnp
from jax import lax
from jax.experimental import pallas as pl
from jax.experimental.pallas import tpu as pltpu
from jax.experimental.pallas import tpu_sc as plsc

NC = 2
NS = 16
NW = NC * NS
LANES = 16
GRID_R = 64


@functools.partial(jax.jit, static_argnums=(9, 10, 11))
def _sc_loss(e0, e1, refx, refy, refz, gridf, vxh, vyh, vzh, N, C, CV):
    E = e0.shape[0]
    EPT = E // NW
    NCH = EPT // C
    NPAIR = (NCH - 1) // 2
    NP = CV * NW
    RPT = NP // NS
    BLK = 224
    assert RPT % BLK == 0
    NBLK = RPT // BLK

    mesh = plsc.VectorSubcoreMesh(
        core_axis_name="c", subcore_axis_name="s",
        num_cores=NC, num_subcores=NS)

    W = 8

    def edge_set():
        return (
            [pltpu.VMEM((C,), jnp.int32) for _ in range(2)]
            + [pltpu.VMEM((C, W), jnp.float32) for _ in range(2)]
            + [pltpu.VMEM((C,), jnp.float32) for _ in range(3)]
        )

    scratch = edge_set() + edge_set()
    scratch += [pltpu.VMEM((BLK,), jnp.float32) for _ in range(3)]
    scratch += [pltpu.VMEM((BLK, W), jnp.float32)]
    H = CV // 2
    Q = CV // 4
    scratch += [pltpu.VMEM((H,), jnp.float32) for _ in range(6)]
    scratch += [pltpu.VMEM((Q,), jnp.int32) for _ in range(8)]
    scratch += [pltpu.VMEM((Q,), jnp.float32) for _ in range(8)]
    scratch += [
        pltpu.VMEM((LANES,), jnp.float32),
        pltpu.SemaphoreType.DMA,
        pltpu.SemaphoreType.DMA,
        pltpu.SemaphoreType.DMA,
    ]

    @functools.partial(
        pl.kernel,
        out_type=(jax.ShapeDtypeStruct((NW, LANES), jnp.float32),
                  jax.ShapeDtypeStruct((NP, W), jnp.float32)),
        mesh=mesh,
        scratch_types=scratch,
        compiler_params=pltpu.CompilerParams(
            needs_layout_passes=False, use_tc_tiling_on_sc=False),
    )
    def k(e0_h, e1_h, rx_h, ry_h, rz_h, gridf_h, vx_h, vy_h, vz_h,
          out_h, tab_h, *scr):
        bufA = scr[0:7]
        bufB = scr[7:14]
        bvx, bvy, bvz = scr[14:17]
        stag = scr[17]
        vh = scr[18:24]
        cidx = scr[24:32]
        cval = scr[32:40]
        acc_v = scr[40]
        semG, semI, semC = scr[41:44]

        sid = lax.axis_index("s")
        wid = sid * NC + lax.axis_index("c")
        iot = lax.iota(jnp.int32, LANES)
        colx = iot * 0
        coly = colx + 1
        colz = colx + 2
        eb = wid * EPT

        tbase = sid * RPT

        def build_blk(b, carry):
            rb = tbase + b * BLK
            cb0 = pltpu.async_copy(vx_h.at[pl.ds(rb, BLK)], bvx, semI)
            cb1 = pltpu.async_copy(vy_h.at[pl.ds(rb, BLK)], bvy, semI)
            cb2 = pltpu.async_copy(vz_h.at[pl.ds(rb, BLK)], bvz, semI)
            cb0.wait()
            cb1.wait()
            cb2.wait()

            def grp(g, c):
                sl = pl.ds(g * LANES, LANES)
                srow = g * LANES + iot
                plsc.store_scatter(stag, [srow, colx], bvx[sl])
                plsc.store_scatter(stag, [srow, coly], bvy[sl])
                plsc.store_scatter(stag, [srow, colz], bvz[sl])
                return c

            lax.fori_loop(0, BLK // LANES, grp, 0)
            pltpu.sync_copy(stag, tab_h.at[pl.ds(rb, BLK)])
            return carry

        lax.fori_loop(0, NBLK, build_blk, 0)
        plsc.subcore_barrier()

        def fire_idx(buf, base):
            pltpu.async_copy(e0_h.at[pl.ds(base, C)], buf[0], semI)
            pltpu.async_copy(e1_h.at[pl.ds(base, C)], buf[1], semI)

        def wait_idx(buf):
            pltpu.make_async_copy(e0_h.at[pl.ds(0, C)], buf[0], semI).wait()
            pltpu.make_async_copy(e1_h.at[pl.ds(0, C)], buf[1], semI).wait()

        def fire_gathers(buf, base):
            pltpu.async_copy(tab_h.at[buf[0]], buf[2], semG)
            pltpu.async_copy(tab_h.at[buf[1]], buf[3], semG)
            sl = pl.ds(base, C)
            pltpu.async_copy(rx_h.at[sl], buf[4], semG)
            pltpu.async_copy(ry_h.at[sl], buf[5], semG)
            pltpu.async_copy(rz_h.at[sl], buf[6], semG)

        def wait_gathers(buf):
            for j in (2, 3):
                pltpu.make_async_copy(tab_h.at[buf[0]], buf[j], semG).wait()
            for j in (4, 5, 6):
                pltpu.make_async_copy(rx_h.at[pl.ds(0, C)], buf[j], semG).wait()

        def compute(buf, acc):
            r0, r1 = buf[2], buf[3]

            def grp(g, a):
                s = pl.ds(g * LANES, LANES)
                row = g * LANES + iot
                dx = (plsc.load_gather(r0, [row, colx])
                      - plsc.load_gather(r1, [row, colx]) - buf[4][s])
                dy = (plsc.load_gather(r0, [row, coly])
                      - plsc.load_gather(r1, [row, coly]) - buf[5][s])
                dz = (plsc.load_gather(r0, [row, colz])
                      - plsc.load_gather(r1, [row, colz]) - buf[6][s])
                return a + (dx * dx + dy * dy + dz * dz)

            return lax.fori_loop(0, C // LANES, grp, acc)

        fire_idx(bufA, eb)

        vb = wid * CV
        cvl = [
            pltpu.async_copy(vx_h.at[pl.ds(vb, H)], vh[0], semC),
            pltpu.async_copy(vy_h.at[pl.ds(vb, H)], vh[1], semC),
            pltpu.async_copy(vz_h.at[pl.ds(vb, H)], vh[2], semC),
            pltpu.async_copy(vx_h.at[pl.ds(vb + H, H)], vh[3], semC),
            pltpu.async_copy(vy_h.at[pl.ds(vb + H, H)], vh[4], semC),
            pltpu.async_copy(vz_h.at[pl.ds(vb + H, H)], vh[5], semC),
        ]
        for cp in cvl:
            cp.wait()

        def prep(p):
            u = jnp.minimum(
                jnp.maximum((p + 1.0) * 0.5 * float(GRID_R - 1), 0.0),
                float(GRID_R - 1) - 1e-4)
            i0 = u.astype(jnp.int32)
            return i0, u - i0.astype(jnp.float32)

        def vprep(half, qp, g, carry):
            sl = pl.ds(g * LANES, LANES)
            src = pl.ds(qp * Q + g * LANES, LANES)
            x0, fx = prep(vh[3 * half + 0][src])
            y0, fy = prep(vh[3 * half + 1][src])
            z0, fz = prep(vh[3 * half + 2][src])
            b = x0 * (GRID_R * GRID_R) + y0 * GRID_R + z0
            cidx[0][sl] = b
            cidx[1][sl] = b + 1
            cidx[2][sl] = b + GRID_R
            cidx[3][sl] = b + GRID_R + 1
            cidx[4][sl] = b + GRID_R * GRID_R
            cidx[5][sl] = b + GRID_R * GRID_R + 1
            cidx[6][sl] = b + GRID_R * GRID_R + GRID_R
            cidx[7][sl] = b + GRID_R * GRID_R + GRID_R + 1
            return carry

        def fire_corners(half, qp):
            lax.fori_loop(0, Q // LANES, functools.partial(vprep, half, qp), 0)
            for j in range(8):
                pltpu.async_copy(gridf_h.at[cidx[j]], cval[j], semC)

        fire_corners(0, 0)

        wait_idx(bufA)
        fire_gathers(bufA, eb)
        wait_gathers(bufA)
        acc = compute(bufA, jnp.zeros((LANES,), jnp.float32))
        if NCH > 1:
            fire_idx(bufA, eb + C)
            wait_idx(bufA)
            fire_gathers(bufA, eb + C)

        def pair(j, acc):
            baseA = eb + (2 * j + 1) * C
            baseB = baseA + C
            fire_idx(bufB, baseB)
            wait_idx(bufB)
            fire_gathers(bufB, baseB)
            wait_gathers(bufA)
            acc = compute(bufA, acc)

            @pl.when(j + 1 < NPAIR)
            def _():
                fire_idx(bufA, baseB + C)
                wait_idx(bufA)
                fire_gathers(bufA, baseB + C)

            wait_gathers(bufB)
            return compute(bufB, acc)

        acc = lax.fori_loop(0, NPAIR, pair, acc)

        def vgrp(half, qp, g, a):
            sl = pl.ds(g * LANES, LANES)
            src = pl.ds(qp * Q + g * LANES, LANES)
            _, fx = prep(vh[3 * half + 0][src])
            _, fy = prep(vh[3 * half + 1][src])
            _, fz = prep(vh[3 * half + 2][src])
            c00 = cval[0][sl] * (1 - fx) + cval[4][sl] * fx
            c10 = cval[2][sl] * (1 - fx) + cval[6][sl] * fx
            c01 = cval[1][sl] * (1 - fx) + cval[5][sl] * fx
            c11 = cval[3][sl] * (1 - fx) + cval[7][sl] * fx
            c0 = c00 * (1 - fy) + c10 * fy
            c1 = c01 * (1 - fy) + c11 * fy
            d = c0 * (1 - fz) + c1 * fz
            vid = vb + half * H + qp * Q + g * LANES + iot
            dm = jnp.where(vid < N, d, 0.0)
            return a + dm * dm

        def drain_corners(half, qp, a):
            for j in range(8):
                pltpu.make_async_copy(gridf_h.at[cidx[j]], cval[j], semC).wait()
            return lax.fori_loop(0, Q // LANES,
                                 functools.partial(vgrp, half, qp), a)

        acc = drain_corners(0, 0, acc)
        for half, qp in ((0, 1), (1, 0), (1, 1)):
            fire_corners(half, qp)
            acc = drain_corners(half, qp, acc)

        acc_v[...] = acc
        pltpu.sync_copy(acc_v, out_h.at[wid])

    return k(e0, e1, refx, refy, refz, gridf, vxh, vyh, vzh)[0]


def kernel(src_V, src_E, dist_grid, ref_edge_vec):
    N = src_V.shape[0]
    E = src_E.shape[0]
    assert E % NW == 0
    ept = E // NW
    C = 16
    for cand in range(16, 2049, 16):
        if ept % cand == 0 and (ept // cand) % 2 == 1:
            C = cand
    align = LANES * NW
    NP = ((N + align - 1) // align) * align
    CV = NP // NW

    e0 = src_E[:, 0]
    e1 = src_E[:, 1]
    refx = ref_edge_vec[:, 0]
    refy = ref_edge_vec[:, 1]
    refz = ref_edge_vec[:, 2]
    gridf = dist_grid.reshape(-1)
    pad = NP - N
    vx = jnp.pad(src_V[:, 0], (0, pad))
    vy = jnp.pad(src_V[:, 1], (0, pad))
    vz = jnp.pad(src_V[:, 2], (0, pad))

    out = _sc_loss(e0, e1, refx, refy, refz, gridf, vx, vy, vz, N, C, CV)
    return 0.5 * jnp.sum(out)


# speedup vs baseline: 20.0226x; 1.0046x over previous
import functools

import jax
import jax.numpy as jnp
from jax import lax
from jax.experimental import pallas as pl
from jax.experimental.pallas import tpu as pltpu
from jax.experimental.pallas import tpu_sc as plsc

NC = 2
NS = 16
NW = NC * NS
LANES = 16
GRID_R = 64


@functools.partial(jax.jit, static_argnums=(9, 10, 11))
def _sc_loss(e0, e1, refx, refy, refz, gridf, vxh, vyh, vzh, N, C, CV):
    E = e0.shape[0]
    EPT = E // NW
    NCH = EPT // C
    NPAIR = (NCH - 1) // 2
    NP = CV * NW
    RPT = NP // NS
    BLK = 224
    assert RPT % BLK == 0
    NBLK = RPT // BLK

    mesh = plsc.VectorSubcoreMesh(
        core_axis_name="c", subcore_axis_name="s",
        num_cores=NC, num_subcores=NS)

    W = 8

    def edge_set():
        return (
            [pltpu.VMEM((C,), jnp.int32) for _ in range(2)]
            + [pltpu.VMEM((C, W), jnp.float32) for _ in range(2)]
            + [pltpu.VMEM((C,), jnp.float32) for _ in range(3)]
        )

    scratch = edge_set() + edge_set()
    scratch += [pltpu.VMEM((BLK,), jnp.float32) for _ in range(3)]
    scratch += [pltpu.VMEM((BLK, W), jnp.float32)]
    H = CV // 2
    Q = CV // 4
    scratch += [pltpu.VMEM((H,), jnp.float32) for _ in range(6)]
    scratch += [pltpu.VMEM((Q,), jnp.int32) for _ in range(8)]
    scratch += [pltpu.VMEM((Q,), jnp.float32) for _ in range(8)]
    scratch += [
        pltpu.VMEM((LANES,), jnp.float32),
        pltpu.SemaphoreType.DMA,
        pltpu.SemaphoreType.DMA,
        pltpu.SemaphoreType.DMA,
        pltpu.SemaphoreType.DMA,
    ]

    @functools.partial(
        pl.kernel,
        out_type=(jax.ShapeDtypeStruct((NW, LANES), jnp.float32),
                  jax.ShapeDtypeStruct((NP, W), jnp.float32)),
        mesh=mesh,
        scratch_types=scratch,
        compiler_params=pltpu.CompilerParams(
            needs_layout_passes=False, use_tc_tiling_on_sc=False),
    )
    def k(e0_h, e1_h, rx_h, ry_h, rz_h, gridf_h, vx_h, vy_h, vz_h,
          out_h, tab_h, *scr):
        bufA = scr[0:7]
        bufB = scr[7:14]
        bvx, bvy, bvz = scr[14:17]
        stag = scr[17]
        vh = scr[18:24]
        cidx = scr[24:32]
        cval = scr[32:40]
        acc_v = scr[40]
        semG, semI, semC, semB = scr[41:45]

        sid = lax.axis_index("s")
        wid = sid * NC + lax.axis_index("c")
        iot = lax.iota(jnp.int32, LANES)
        colx = iot * 0
        coly = colx + 1
        colz = colx + 2
        eb = wid * EPT

        eb = wid * EPT
        vb = wid * CV
        pltpu.async_copy(e0_h.at[pl.ds(eb, C)], bufA[0], semI)
        pltpu.async_copy(e1_h.at[pl.ds(eb, C)], bufA[1], semI)
        cvl = [
            pltpu.async_copy(vx_h.at[pl.ds(vb, H)], vh[0], semC),
            pltpu.async_copy(vy_h.at[pl.ds(vb, H)], vh[1], semC),
            pltpu.async_copy(vz_h.at[pl.ds(vb, H)], vh[2], semC),
            pltpu.async_copy(vx_h.at[pl.ds(vb + H, H)], vh[3], semC),
            pltpu.async_copy(vy_h.at[pl.ds(vb + H, H)], vh[4], semC),
            pltpu.async_copy(vz_h.at[pl.ds(vb + H, H)], vh[5], semC),
        ]

        tbase = sid * RPT

        def build_blk(b, carry):
            rb = tbase + b * BLK
            cb0 = pltpu.async_copy(vx_h.at[pl.ds(rb, BLK)], bvx, semB)
            cb1 = pltpu.async_copy(vy_h.at[pl.ds(rb, BLK)], bvy, semB)
            cb2 = pltpu.async_copy(vz_h.at[pl.ds(rb, BLK)], bvz, semB)
            cb0.wait()
            cb1.wait()
            cb2.wait()

            def grp(g, c):
                sl = pl.ds(g * LANES, LANES)
                srow = g * LANES + iot
                plsc.store_scatter(stag, [srow, colx], bvx[sl])
                plsc.store_scatter(stag, [srow, coly], bvy[sl])
                plsc.store_scatter(stag, [srow, colz], bvz[sl])
                return c

            lax.fori_loop(0, BLK // LANES, grp, 0)
            pltpu.sync_copy(stag, tab_h.at[pl.ds(rb, BLK)])
            return carry

        lax.fori_loop(0, NBLK, build_blk, 0)
        plsc.subcore_barrier()

        def fire_idx(buf, base):
            pltpu.async_copy(e0_h.at[pl.ds(base, C)], buf[0], semI)
            pltpu.async_copy(e1_h.at[pl.ds(base, C)], buf[1], semI)

        def wait_idx(buf):
            pltpu.make_async_copy(e0_h.at[pl.ds(0, C)], buf[0], semI).wait()
            pltpu.make_async_copy(e1_h.at[pl.ds(0, C)], buf[1], semI).wait()

        def fire_gathers(buf, base):
            pltpu.async_copy(tab_h.at[buf[0]], buf[2], semG)
            pltpu.async_copy(tab_h.at[buf[1]], buf[3], semG)
            sl = pl.ds(base, C)
            pltpu.async_copy(rx_h.at[sl], buf[4], semG)
            pltpu.async_copy(ry_h.at[sl], buf[5], semG)
            pltpu.async_copy(rz_h.at[sl], buf[6], semG)

        def wait_gathers(buf):
            for j in (2, 3):
                pltpu.make_async_copy(tab_h.at[buf[0]], buf[j], semG).wait()
            for j in (4, 5, 6):
                pltpu.make_async_copy(rx_h.at[pl.ds(0, C)], buf[j], semG).wait()

        def compute(buf, acc):
            r0, r1 = buf[2], buf[3]

            def grp(g, a):
                s = pl.ds(g * LANES, LANES)
                row = g * LANES + iot
                dx = (plsc.load_gather(r0, [row, colx])
                      - plsc.load_gather(r1, [row, colx]) - buf[4][s])
                dy = (plsc.load_gather(r0, [row, coly])
                      - plsc.load_gather(r1, [row, coly]) - buf[5][s])
                dz = (plsc.load_gather(r0, [row, colz])
                      - plsc.load_gather(r1, [row, colz]) - buf[6][s])
                return a + (dx * dx + dy * dy + dz * dz)

            return lax.fori_loop(0, C // LANES, grp, acc)

        for cp in cvl:
            cp.wait()

        def prep(p):
            u = jnp.minimum(
                jnp.maximum((p + 1.0) * 0.5 * float(GRID_R - 1), 0.0),
                float(GRID_R - 1) - 1e-4)
            i0 = u.astype(jnp.int32)
            return i0, u - i0.astype(jnp.float32)

        def vprep(half, qp, g, carry):
            sl = pl.ds(g * LANES, LANES)
            src = pl.ds(qp * Q + g * LANES, LANES)
            x0, fx = prep(vh[3 * half + 0][src])
            y0, fy = prep(vh[3 * half + 1][src])
            z0, fz = prep(vh[3 * half + 2][src])
            b = x0 * (GRID_R * GRID_R) + y0 * GRID_R + z0
            cidx[0][sl] = b
            cidx[1][sl] = b + 1
            cidx[2][sl] = b + GRID_R
            cidx[3][sl] = b + GRID_R + 1
            cidx[4][sl] = b + GRID_R * GRID_R
            cidx[5][sl] = b + GRID_R * GRID_R + 1
            cidx[6][sl] = b + GRID_R * GRID_R + GRID_R
            cidx[7][sl] = b + GRID_R * GRID_R + GRID_R + 1
            return carry

        def fire_corners(half, qp):
            lax.fori_loop(0, Q // LANES, functools.partial(vprep, half, qp), 0)
            for j in range(8):
                pltpu.async_copy(gridf_h.at[cidx[j]], cval[j], semC)

        fire_corners(0, 0)

        wait_idx(bufA)
        fire_gathers(bufA, eb)
        wait_gathers(bufA)
        acc = compute(bufA, jnp.zeros((LANES,), jnp.float32))
        if NCH > 1:
            fire_idx(bufA, eb + C)
            wait_idx(bufA)
            fire_gathers(bufA, eb + C)

        def pair(j, acc):
            baseA = eb + (2 * j + 1) * C
            baseB = baseA + C
            fire_idx(bufB, baseB)
            wait_idx(bufB)
            fire_gathers(bufB, baseB)
            wait_gathers(bufA)
            acc = compute(bufA, acc)

            @pl.when(j + 1 < NPAIR)
            def _():
                fire_idx(bufA, baseB + C)
                wait_idx(bufA)
                fire_gathers(bufA, baseB + C)

            wait_gathers(bufB)
            return compute(bufB, acc)

        acc = lax.fori_loop(0, NPAIR, pair, acc)

        def vgrp(half, qp, g, a):
            sl = pl.ds(g * LANES, LANES)
            src = pl.ds(qp * Q + g * LANES, LANES)
            _, fx = prep(vh[3 * half + 0][src])
            _, fy = prep(vh[3 * half + 1][src])
            _, fz = prep(vh[3 * half + 2][src])
            c00 = cval[0][sl] * (1 - fx) + cval[4][sl] * fx
            c10 = cval[2][sl] * (1 - fx) + cval[6][sl] * fx
            c01 = cval[1][sl] * (1 - fx) + cval[5][sl] * fx
            c11 = cval[3][sl] * (1 - fx) + cval[7][sl] * fx
            c0 = c00 * (1 - fy) + c10 * fy
            c1 = c01 * (1 - fy) + c11 * fy
            d = c0 * (1 - fz) + c1 * fz
            vid = vb + half * H + qp * Q + g * LANES + iot
            dm = jnp.where(vid < N, d, 0.0)
            return a + dm * dm

        def drain_corners(half, qp, a):
            for j in range(8):
                pltpu.make_async_copy(gridf_h.at[cidx[j]], cval[j], semC).wait()
            return lax.fori_loop(0, Q // LANES,
                                 functools.partial(vgrp, half, qp), a)

        acc = drain_corners(0, 0, acc)
        for half, qp in ((0, 1), (1, 0), (1, 1)):
            fire_corners(half, qp)
            acc = drain_corners(half, qp, acc)

        acc_v[...] = acc
        pltpu.sync_copy(acc_v, out_h.at[wid])

    return k(e0, e1, refx, refy, refz, gridf, vxh, vyh, vzh)[0]


def kernel(src_V, src_E, dist_grid, ref_edge_vec):
    N = src_V.shape[0]
    E = src_E.shape[0]
    assert E % NW == 0
    ept = E // NW
    C = 16
    for cand in range(16, 2049, 16):
        if ept % cand == 0 and (ept // cand) % 2 == 1:
            C = cand
    align = LANES * NW
    NP = ((N + align - 1) // align) * align
    CV = NP // NW

    e0 = src_E[:, 0]
    e1 = src_E[:, 1]
    refx = ref_edge_vec[:, 0]
    refy = ref_edge_vec[:, 1]
    refz = ref_edge_vec[:, 2]
    gridf = dist_grid.reshape(-1)
    pad = NP - N
    vx = jnp.pad(src_V[:, 0], (0, pad))
    vy = jnp.pad(src_V[:, 1], (0, pad))
    vz = jnp.pad(src_V[:, 2], (0, pad))

    out = _sc_loss(e0, e1, refx, refy, refz, gridf, vx, vy, vz, N, C, CV)
    return 0.5 * jnp.sum(out)


# speedup vs baseline: 20.6214x; 1.0299x over previous
import functools

import jax
import jax.numpy as jnp
from jax import lax
from jax.experimental import pallas as pl
from jax.experimental.pallas import tpu as pltpu
from jax.experimental.pallas import tpu_sc as plsc

NC = 2
NS = 16
NW = NC * NS
LANES = 16
GRID_R = 64


@functools.partial(jax.jit, static_argnums=(9, 10, 11))
def _sc_loss(e0, e1, refx, refy, refz, gridf, vxh, vyh, vzh, N, C, CV):
    E = e0.shape[0]
    EPT = E // NW
    NCH = EPT // C
    NPAIR = (NCH - 1) // 2
    NP = CV * NW
    RPT = NP // NS
    BLK = 224
    assert RPT % BLK == 0
    NBLK = RPT // BLK

    mesh = plsc.VectorSubcoreMesh(
        core_axis_name="c", subcore_axis_name="s",
        num_cores=NC, num_subcores=NS)

    W = 8

    def edge_set():
        return (
            [pltpu.VMEM((C,), jnp.int32) for _ in range(2)]
            + [pltpu.VMEM((C, W), jnp.float32) for _ in range(2)]
            + [pltpu.VMEM((C,), jnp.float32) for _ in range(3)]
        )

    scratch = edge_set() + edge_set()
    scratch += [pltpu.VMEM((BLK,), jnp.float32) for _ in range(6)]
    scratch += [pltpu.VMEM((BLK, W), jnp.float32) for _ in range(2)]
    H = CV // 2
    Q = CV // 4
    scratch += [pltpu.VMEM((H,), jnp.float32) for _ in range(6)]
    scratch += [pltpu.VMEM((Q,), jnp.int32) for _ in range(8)]
    scratch += [pltpu.VMEM((Q,), jnp.float32) for _ in range(8)]
    scratch += [
        pltpu.VMEM((LANES,), jnp.float32),
        pltpu.SemaphoreType.DMA,
        pltpu.SemaphoreType.DMA,
        pltpu.SemaphoreType.DMA,
        pltpu.SemaphoreType.DMA,
        pltpu.SemaphoreType.DMA,
    ]

    @functools.partial(
        pl.kernel,
        out_type=(jax.ShapeDtypeStruct((NW, LANES), jnp.float32),
                  jax.ShapeDtypeStruct((NP, W), jnp.float32)),
        mesh=mesh,
        scratch_types=scratch,
        compiler_params=pltpu.CompilerParams(
            needs_layout_passes=False, use_tc_tiling_on_sc=False),
    )
    def k(e0_h, e1_h, rx_h, ry_h, rz_h, gridf_h, vx_h, vy_h, vz_h,
          out_h, tab_h, *scr):
        bufA = scr[0:7]
        bufB = scr[7:14]
        bsets = (scr[14:17], scr[17:20])
        stags = scr[20:22]
        vh = scr[22:28]
        cidx = scr[28:36]
        cval = scr[36:44]
        acc_v = scr[44]
        semG, semI, semC, semB, semB2 = scr[45:50]

        sid = lax.axis_index("s")
        wid = sid * NC + lax.axis_index("c")
        iot = lax.iota(jnp.int32, LANES)
        colx = iot * 0
        coly = colx + 1
        colz = colx + 2
        eb = wid * EPT

        eb = wid * EPT
        vb = wid * CV
        pltpu.async_copy(e0_h.at[pl.ds(eb, C)], bufA[0], semI)
        pltpu.async_copy(e1_h.at[pl.ds(eb, C)], bufA[1], semI)
        cvl = [
            pltpu.async_copy(vx_h.at[pl.ds(vb, H)], vh[0], semC),
            pltpu.async_copy(vy_h.at[pl.ds(vb, H)], vh[1], semC),
            pltpu.async_copy(vz_h.at[pl.ds(vb, H)], vh[2], semC),
            pltpu.async_copy(vx_h.at[pl.ds(vb + H, H)], vh[3], semC),
            pltpu.async_copy(vy_h.at[pl.ds(vb + H, H)], vh[4], semC),
            pltpu.async_copy(vz_h.at[pl.ds(vb + H, H)], vh[5], semC),
        ]

        tbase = sid * RPT

        def bfire_loads(p, b):
            rb = tbase + b * BLK
            pltpu.async_copy(vx_h.at[pl.ds(rb, BLK)], bsets[p][0], semB)
            pltpu.async_copy(vy_h.at[pl.ds(rb, BLK)], bsets[p][1], semB)
            pltpu.async_copy(vz_h.at[pl.ds(rb, BLK)], bsets[p][2], semB)

        def bwait_loads(p):
            for r in range(3):
                pltpu.make_async_copy(
                    vx_h.at[pl.ds(0, BLK)], bsets[p][r], semB).wait()

        bfire_loads(0, 0)
        bfire_loads(1, 1)

        def build_pair(j, carry):
            for p in range(2):
                b = 2 * j + p
                bvx, bvy, bvz = bsets[p]
                stag = stags[p]
                bwait_loads(p)

                @pl.when(j > 0)
                def _():
                    pltpu.make_async_copy(
                        stag, tab_h.at[pl.ds(tbase, BLK)], semB2).wait()

                def grp(g, c):
                    sl = pl.ds(g * LANES, LANES)
                    srow = g * LANES + iot
                    plsc.store_scatter(stag, [srow, colx], bvx[sl])
                    plsc.store_scatter(stag, [srow, coly], bvy[sl])
                    plsc.store_scatter(stag, [srow, colz], bvz[sl])
                    return c

                lax.fori_loop(0, BLK // LANES, grp, 0)
                pltpu.async_copy(stag, tab_h.at[pl.ds(tbase + b * BLK, BLK)],
                                 semB2)

                @pl.when(b + 2 < NBLK)
                def _():
                    bfire_loads(p, b + 2)
            return carry

        assert NBLK % 2 == 0
        lax.fori_loop(0, NBLK // 2, build_pair, 0)
        for p in range(2):
            pltpu.make_async_copy(
                stags[p], tab_h.at[pl.ds(tbase, BLK)], semB2).wait()
        plsc.subcore_barrier()

        def fire_idx(buf, base):
            pltpu.async_copy(e0_h.at[pl.ds(base, C)], buf[0], semI)
            pltpu.async_copy(e1_h.at[pl.ds(base, C)], buf[1], semI)

        def wait_idx(buf):
            pltpu.make_async_copy(e0_h.at[pl.ds(0, C)], buf[0], semI).wait()
            pltpu.make_async_copy(e1_h.at[pl.ds(0, C)], buf[1], semI).wait()

        def fire_gathers(buf, base):
            pltpu.async_copy(tab_h.at[buf[0]], buf[2], semG)
            pltpu.async_copy(tab_h.at[buf[1]], buf[3], semG)
            sl = pl.ds(base, C)
            pltpu.async_copy(rx_h.at[sl], buf[4], semG)
            pltpu.async_copy(ry_h.at[sl], buf[5], semG)
            pltpu.async_copy(rz_h.at[sl], buf[6], semG)

        def wait_gathers(buf):
            for j in (2, 3):
                pltpu.make_async_copy(tab_h.at[buf[0]], buf[j], semG).wait()
            for j in (4, 5, 6):
                pltpu.make_async_copy(rx_h.at[pl.ds(0, C)], buf[j], semG).wait()

        def compute(buf, acc):
            r0, r1 = buf[2], buf[3]

            def grp(g, a):
                s = pl.ds(g * LANES, LANES)
                row = g * LANES + iot
                dx = (plsc.load_gather(r0, [row, colx])
                      - plsc.load_gather(r1, [row, colx]) - buf[4][s])
                dy = (plsc.load_gather(r0, [row, coly])
                      - plsc.load_gather(r1, [row, coly]) - buf[5][s])
                dz = (plsc.load_gather(r0, [row, colz])
                      - plsc.load_gather(r1, [row, colz]) - buf[6][s])
                return a + (dx * dx + dy * dy + dz * dz)

            return lax.fori_loop(0, C // LANES, grp, acc)

        for cp in cvl:
            cp.wait()

        def prep(p):
            u = jnp.minimum(
                jnp.maximum((p + 1.0) * 0.5 * float(GRID_R - 1), 0.0),
                float(GRID_R - 1) - 1e-4)
            i0 = u.astype(jnp.int32)
            return i0, u - i0.astype(jnp.float32)

        def vprep(half, qp, g, carry):
            sl = pl.ds(g * LANES, LANES)
            src = pl.ds(qp * Q + g * LANES, LANES)
            x0, fx = prep(vh[3 * half + 0][src])
            y0, fy = prep(vh[3 * half + 1][src])
            z0, fz = prep(vh[3 * half + 2][src])
            b = x0 * (GRID_R * GRID_R) + y0 * GRID_R + z0
            cidx[0][sl] = b
            cidx[1][sl] = b + 1
            cidx[2][sl] = b + GRID_R
            cidx[3][sl] = b + GRID_R + 1
            cidx[4][sl] = b + GRID_R * GRID_R
            cidx[5][sl] = b + GRID_R * GRID_R + 1
            cidx[6][sl] = b + GRID_R * GRID_R + GRID_R
            cidx[7][sl] = b + GRID_R * GRID_R + GRID_R + 1
            return carry

        def fire_corners(half, qp):
            lax.fori_loop(0, Q // LANES, functools.partial(vprep, half, qp), 0)
            for j in range(8):
                pltpu.async_copy(gridf_h.at[cidx[j]], cval[j], semC)

        fire_corners(0, 0)

        wait_idx(bufA)
        fire_gathers(bufA, eb)
        wait_gathers(bufA)
        acc = compute(bufA, jnp.zeros((LANES,), jnp.float32))
        if NCH > 1:
            fire_idx(bufA, eb + C)
            wait_idx(bufA)
            fire_gathers(bufA, eb + C)

        def pair(j, acc):
            baseA = eb + (2 * j + 1) * C
            baseB = baseA + C
            fire_idx(bufB, baseB)
            wait_idx(bufB)
            fire_gathers(bufB, baseB)
            wait_gathers(bufA)
            acc = compute(bufA, acc)

            @pl.when(j + 1 < NPAIR)
            def _():
                fire_idx(bufA, baseB + C)
                wait_idx(bufA)
                fire_gathers(bufA, baseB + C)

            wait_gathers(bufB)
            return compute(bufB, acc)

        acc = lax.fori_loop(0, NPAIR, pair, acc)

        def vgrp(half, qp, g, a):
            sl = pl.ds(g * LANES, LANES)
            src = pl.ds(qp * Q + g * LANES, LANES)
            _, fx = prep(vh[3 * half + 0][src])
            _, fy = prep(vh[3 * half + 1][src])
            _, fz = prep(vh[3 * half + 2][src])
            c00 = cval[0][sl] * (1 - fx) + cval[4][sl] * fx
            c10 = cval[2][sl] * (1 - fx) + cval[6][sl] * fx
            c01 = cval[1][sl] * (1 - fx) + cval[5][sl] * fx
            c11 = cval[3][sl] * (1 - fx) + cval[7][sl] * fx
            c0 = c00 * (1 - fy) + c10 * fy
            c1 = c01 * (1 - fy) + c11 * fy
            d = c0 * (1 - fz) + c1 * fz
            vid = vb + half * H + qp * Q + g * LANES + iot
            dm = jnp.where(vid < N, d, 0.0)
            return a + dm * dm

        def drain_corners(half, qp, a):
            for j in range(8):
                pltpu.make_async_copy(gridf_h.at[cidx[j]], cval[j], semC).wait()
            return lax.fori_loop(0, Q // LANES,
                                 functools.partial(vgrp, half, qp), a)

        acc = drain_corners(0, 0, acc)
        for half, qp in ((0, 1), (1, 0), (1, 1)):
            fire_corners(half, qp)
            acc = drain_corners(half, qp, acc)

        acc_v[...] = acc
        pltpu.sync_copy(acc_v, out_h.at[wid])

    return k(e0, e1, refx, refy, refz, gridf, vxh, vyh, vzh)[0]


def kernel(src_V, src_E, dist_grid, ref_edge_vec):
    N = src_V.shape[0]
    E = src_E.shape[0]
    assert E % NW == 0
    ept = E // NW
    C = 16
    for cand in range(16, 2049, 16):
        if ept % cand == 0 and (ept // cand) % 2 == 1:
            C = cand
    align = LANES * NW
    NP = ((N + align - 1) // align) * align
    CV = NP // NW

    e0 = src_E[:, 0]
    e1 = src_E[:, 1]
    refx = ref_edge_vec[:, 0]
    refy = ref_edge_vec[:, 1]
    refz = ref_edge_vec[:, 2]
    gridf = dist_grid.reshape(-1)
    pad = NP - N
    vx = jnp.pad(src_V[:, 0], (0, pad))
    vy = jnp.pad(src_V[:, 1], (0, pad))
    vz = jnp.pad(src_V[:, 2], (0, pad))

    out = _sc_loss(e0, e1, refx, refy, refz, gridf, vx, vy, vz, N, C, CV)
    return 0.5 * jnp.sum(out)
